# baseline (speedup 1.0000x reference)
import functools

import jax
import jax.numpy as jnp
from jax import lax
from jax.experimental import pallas as pl
from jax.experimental.pallas import tpu as pltpu
from jax.experimental.pallas import tpu_sc as plsc

N_NODES = 10000
N_EDGES = 320000
N_GRAPHS = 64
D_IN = 128
D_HALF = 64
D_HID = 16

NC, NS = 2, 16
NW = NC * NS
R = 10112
RS = 10240
TRASH = R
CH1 = 512
NCH2 = 40
BLK = 8
CH2 = 1024
NCH = 10
E_PAD = NS * NCH2 * CH1
ZROWS = RS // NS
OROWS = R // NS

_mesh = plsc.VectorSubcoreMesh(core_axis_name="c", subcore_axis_name="s")
_sc_params = pltpu.CompilerParams(use_tc_tiling_on_sc=False)


def _fill(buf, nrows, width, value):
    vec = jnp.full((16,), value, jnp.float32)

    def row(r, carry):
        for k in range(width // 16):
            buf[r, pl.ds(k * 16, 16)] = vec
        return carry

    lax.fori_loop(0, nrows, row, 0)


def _zero_spmem(stage, spmem, s, width):
    _fill(stage, 128, width, 0.0)
    for z in range(ZROWS // 128):
        pltpu.sync_copy(stage.at[pl.ds(0, 128)],
                        spmem.at[pl.ds(s * ZROWS + z * 128, 128)])


def _sc_agg_x_body(xcat_hbm, src_hbm, dst_hbm, p_hbm, dg_hbm,
                   src_v, dst_v, rows_v, ones_v, agg_s, deg_s, sem_s):
    c = lax.axis_index("c")
    s = lax.axis_index("s")
    _zero_spmem(rows_v, agg_s, s, D_HALF)

    @pl.when(c == 1)
    def _():
        _zero_spmem(ones_v, deg_s, s, D_HID)
        _fill(ones_v, CH1, D_HID, 1.0)

    off = c * N_NODES
    plsc.subcore_barrier()

    def chunk(j, carry):
        jr = lax.rem(j, BLK)

        @pl.when(jr == 0)
        def _():
            pltpu.sync_copy(src_hbm.at[s, pl.ds(j, BLK)], src_v)
            pltpu.sync_copy(dst_hbm.at[s, pl.ds(j, BLK)], dst_v)

            def adj(r, c2):
                for k in range(CH1 // 16):
                    sl = pl.ds(k * 16, 16)
                    src_v[r, sl] = src_v[r, sl] + off
                return c2

            lax.fori_loop(0, BLK, adj, 0)

        pltpu.sync_copy(xcat_hbm.at[src_v.at[jr]], rows_v)
        pltpu.async_copy(rows_v, agg_s.at[dst_v.at[jr]], sem_s, add=True)

        @pl.when(c == 1)
        def _():
            pltpu.sync_copy(ones_v, deg_s.at[dst_v.at[jr]], add=True)

        pltpu.make_async_copy(rows_v, agg_s.at[dst_v.at[jr]], sem_s).wait()
        return carry

    lax.fori_loop(0, NCH2, chunk, 0)
    plsc.subcore_barrier()
    pltpu.sync_copy(agg_s.at[pl.ds(s * OROWS, OROWS)],
                    p_hbm.at[c, pl.ds(s * OROWS, OROWS)])

    @pl.when(c == 1)
    def _():
        pltpu.sync_copy(deg_s.at[pl.ds(s * OROWS, OROWS)],
                        dg_hbm.at[pl.ds(s * OROWS, OROWS)])


_sc_agg_x = functools.partial(
    pl.kernel,
    mesh=_mesh,
    compiler_params=_sc_params,
    out_type=[jax.ShapeDtypeStruct((NC, R, D_HALF), jnp.float32),
              jax.ShapeDtypeStruct((R, D_HID), jnp.float32)],
    scratch_types=[
        pltpu.VMEM((BLK, CH1), jnp.int32),
        pltpu.VMEM((BLK, CH1), jnp.int32),
        pltpu.VMEM((CH1, D_HALF), jnp.float32),
        pltpu.VMEM((CH1, D_HID), jnp.float32),
        pltpu.VMEM_SHARED((RS, D_HALF), jnp.float32),
        pltpu.VMEM_SHARED((RS, D_HID), jnp.float32),
        pltpu.SemaphoreType.DMA,
    ],
)(_sc_agg_x_body)


def _sc_agg_h_body(h_hbm, src_hbm, dst_hbm, p_hbm,
                   src_v, dst_v, rows_a, rows_b, agg_s,
                   sem_a, sem_b, sem_sa, sem_sb):
    c = lax.axis_index("c")
    s = lax.axis_index("s")
    wid = c * NS + s
    _zero_spmem(rows_a, agg_s, s, D_HID)
    pltpu.sync_copy(src_hbm.at[wid], src_v)
    pltpu.sync_copy(dst_hbm.at[wid], dst_v)
    plsc.subcore_barrier()

    def start(j, buf, sem):
        pltpu.async_copy(h_hbm.at[src_v.at[j]], buf, sem)

    def wait(j, buf, sem):
        pltpu.make_async_copy(h_hbm.at[src_v.at[j]], buf, sem).wait()

    def s_start(j, buf, sem):
        pltpu.async_copy(buf, agg_s.at[dst_v.at[j]], sem, add=True)

    def s_wait(j, buf, sem):
        pltpu.make_async_copy(buf, agg_s.at[dst_v.at[j]], sem).wait()

    start(0, rows_a, sem_a)

    def pair(i, carry):
        j0 = 2 * i
        wait(j0, rows_a, sem_a)
        start(j0 + 1, rows_b, sem_b)
        s_start(j0, rows_a, sem_sa)
        wait(j0 + 1, rows_b, sem_b)
        s_wait(j0, rows_a, sem_sa)
        start(j0 + 2, rows_a, sem_a)
        s_start(j0 + 1, rows_b, sem_sb)
        s_wait(j0 + 1, rows_b, sem_sb)
        return carry

    lax.fori_loop(0, NCH // 2 - 1, pair, 0)
    wait(NCH - 2, rows_a, sem_a)
    start(NCH - 1, rows_b, sem_b)
    s_start(NCH - 2, rows_a, sem_sa)
    wait(NCH - 1, rows_b, sem_b)
    s_wait(NCH - 2, rows_a, sem_sa)
    s_start(NCH - 1, rows_b, sem_sb)
    s_wait(NCH - 1, rows_b, sem_sb)
    plsc.subcore_barrier()
    pltpu.sync_copy(agg_s.at[pl.ds(s * OROWS, OROWS)],
                    p_hbm.at[c, pl.ds(s * OROWS, OROWS)])


_sc_agg_h = functools.partial(
    pl.kernel,
    mesh=_mesh,
    compiler_params=_sc_params,
    out_type=[jax.ShapeDtypeStruct((NC, R, D_HID), jnp.float32)],
    scratch_types=[
        pltpu.VMEM((NCH, CH2), jnp.int32),
        pltpu.VMEM((NCH, CH2), jnp.int32),
        pltpu.VMEM((CH2, D_HID), jnp.float32),
        pltpu.VMEM((CH2, D_HID), jnp.float32),
        pltpu.VMEM_SHARED((RS, D_HID), jnp.float32),
        pltpu.SemaphoreType.DMA,
        pltpu.SemaphoreType.DMA,
        pltpu.SemaphoreType.DMA,
        pltpu.SemaphoreType.DMA,
    ],
)(_sc_agg_h_body)


def _lay1_body(p_ref, d_ref, w1_ref, b1_ref, o_ref):
    agg = jnp.concatenate([p_ref[0], p_ref[1]], axis=1)
    deg = jnp.maximum(d_ref[:, 0:1], 1.0)
    o_ref[...] = jnp.maximum(
        jnp.dot(agg / deg, w1_ref[...],
                preferred_element_type=jnp.float32) + b1_ref[...], 0.0)


def _fin_body(p_ref, d_ref, w2_ref, b2_ref, batch_ref, fcw_ref, fcb_ref,
              o_ref):
    agg = p_ref[0] + p_ref[1]
    deg = jnp.maximum(d_ref[:, 0:1], 1.0)
    h2 = jnp.maximum(
        jnp.dot(agg / deg, w2_ref[...],
                preferred_element_type=jnp.float32) + b2_ref[...], 0.0)
    b = batch_ref[...]
    gid = lax.broadcasted_iota(jnp.int32, (N_GRAPHS, R), 0)
    onehot = (b == gid).astype(jnp.float32)
    sums = jnp.dot(onehot, h2, preferred_element_type=jnp.float32,
                   precision=lax.Precision.HIGHEST)
    counts = jnp.sum(onehot, axis=1, keepdims=True)
    pooled = sums / jnp.maximum(counts, 1.0)
    o_ref[...] = (jnp.dot(pooled, fcw_ref[...],
                          preferred_element_type=jnp.float32) + fcb_ref[...])


_lay1 = pl.pallas_call(
    _lay1_body, out_shape=jax.ShapeDtypeStruct((R, D_HID), jnp.float32))

_fin = pl.pallas_call(
    _fin_body, out_shape=jax.ShapeDtypeStruct((N_GRAPHS, 1), jnp.float32))


def kernel(x, edge_index, batch, W1, b1, W2, b2, fc_W, fc_b):
    src = edge_index[0].astype(jnp.int32)
    dst = edge_index[1].astype(jnp.int32)
    n_pad_e = E_PAD - N_EDGES
    src_p = jnp.concatenate([src, jnp.zeros((n_pad_e,), jnp.int32)])
    dst_p = jnp.concatenate([dst, jnp.full((n_pad_e,), TRASH, jnp.int32)])
    src2 = src_p.reshape(NS, NCH2, CH1)
    dst2 = dst_p.reshape(NS, NCH2, CH1)
    src3 = src_p.reshape(NW, NCH, CH2)
    dst3 = dst_p.reshape(NW, NCH, CH2)
    xcat = jnp.concatenate([x[:, :D_HALF], x[:, D_HALF:]], axis=0)
    batch2 = jnp.concatenate(
        [batch.astype(jnp.int32),
         jnp.full((R - N_NODES,), N_GRAPHS, jnp.int32)]).reshape(1, R)

    p1, dg = _sc_agg_x(xcat, src2, dst2)
    h1 = _lay1(p1, dg, W1, b1.reshape(1, D_HID))
    (p2,) = _sc_agg_h(h1, src3, dst3)
    out = _fin(p2, dg, W2, b2.reshape(1, D_HID), batch2, fc_W,
               fc_b.reshape(1, 1))
    return out


# speedup vs baseline: 1.0763x; 1.0763x over previous
import functools

import jax
import jax.numpy as jnp
from jax import lax
from jax.experimental import pallas as pl
from jax.experimental.pallas import tpu as pltpu
from jax.experimental.pallas import tpu_sc as plsc

N_NODES = 10000
N_EDGES = 320000
N_GRAPHS = 64
D_IN = 128
D_HALF = 64
D_HID = 16

NC, NS = 2, 16
NW = NC * NS
R = 10112
RS = 10240
TRASH = R
CH1 = 256
NCH2 = 80
CH2 = 1024
NCH = 10
E_PAD = NS * NCH2 * CH1
ZROWS = RS // NS
OROWS = R // NS

_mesh = plsc.VectorSubcoreMesh(core_axis_name="c", subcore_axis_name="s")
_sc_params = pltpu.CompilerParams(use_tc_tiling_on_sc=False)


def _fill(buf, nrows, width, value):
    vec = jnp.full((16,), value, jnp.float32)

    def row(r, carry):
        for k in range(width // 16):
            buf[r, pl.ds(k * 16, 16)] = vec
        return carry

    lax.fori_loop(0, nrows, row, 0)


def _zero_spmem(stage, spmem, s, width):
    _fill(stage, 128, width, 0.0)
    for z in range(ZROWS // 128):
        pltpu.sync_copy(stage.at[pl.ds(0, 128)],
                        spmem.at[pl.ds(s * ZROWS + z * 128, 128)])


def _sc_agg_x_body(xcat_hbm, src_hbm, dst_hbm, p_hbm, dg_hbm,
                   src_v, dst_v, rows_a, rows_b, ones_v, agg_s, deg_s,
                   sem_a, sem_b, sem_sa, sem_sb, sem_o):
    c = lax.axis_index("c")
    s = lax.axis_index("s")
    _zero_spmem(rows_a, agg_s, s, D_HALF)

    @pl.when(c == 1)
    def _():
        _zero_spmem(ones_v, deg_s, s, D_HID)
        _fill(ones_v, CH1, D_HID, 1.0)
    pltpu.sync_copy(src_hbm.at[s], src_v)
    pltpu.sync_copy(dst_hbm.at[s], dst_v)
    off = c * N_NODES

    def adj(r, carry):
        for k in range(CH1 // 16):
            sl = pl.ds(k * 16, 16)
            src_v[r, sl] = src_v[r, sl] + off
        return carry

    lax.fori_loop(0, NCH2, adj, 0)
    plsc.subcore_barrier()

    def start(j, buf, sem):
        pltpu.async_copy(xcat_hbm.at[src_v.at[j]], buf, sem)

    def wait(j, buf, sem):
        pltpu.make_async_copy(xcat_hbm.at[src_v.at[j]], buf, sem).wait()

    def s_start(j, buf, sem):
        pltpu.async_copy(buf, agg_s.at[dst_v.at[j]], sem, add=True)

    def s_wait(j, buf, sem):
        pltpu.make_async_copy(buf, agg_s.at[dst_v.at[j]], sem).wait()

    def o_start(j):
        pltpu.async_copy(ones_v, deg_s.at[dst_v.at[j]], sem_o, add=True)

    def o_wait(j):
        pltpu.make_async_copy(ones_v, deg_s.at[dst_v.at[j]], sem_o).wait()

    start(0, rows_a, sem_a)

    def pair(i, carry):
        j0 = 2 * i
        wait(j0, rows_a, sem_a)
        start(j0 + 1, rows_b, sem_b)
        s_start(j0, rows_a, sem_sa)

        @pl.when(c == 1)
        def _():
            o_start(j0)

            @pl.when(i > 0)
            def _():
                o_wait(j0 - 2)

        wait(j0 + 1, rows_b, sem_b)
        s_wait(j0, rows_a, sem_sa)
        start(j0 + 2, rows_a, sem_a)
        s_start(j0 + 1, rows_b, sem_sb)

        @pl.when(c == 1)
        def _():
            o_start(j0 + 1)

            @pl.when(i > 0)
            def _():
                o_wait(j0 - 1)

        s_wait(j0 + 1, rows_b, sem_sb)
        return carry

    lax.fori_loop(0, NCH2 // 2 - 1, pair, 0)
    wait(NCH2 - 2, rows_a, sem_a)
    start(NCH2 - 1, rows_b, sem_b)
    s_start(NCH2 - 2, rows_a, sem_sa)

    @pl.when(c == 1)
    def _():
        o_start(NCH2 - 2)
        o_wait(NCH2 - 4)
        o_start(NCH2 - 1)
        o_wait(NCH2 - 3)

    wait(NCH2 - 1, rows_b, sem_b)
    s_wait(NCH2 - 2, rows_a, sem_sa)
    s_start(NCH2 - 1, rows_b, sem_sb)

    @pl.when(c == 1)
    def _():
        o_wait(NCH2 - 2)
        o_wait(NCH2 - 1)

    s_wait(NCH2 - 1, rows_b, sem_sb)
    plsc.subcore_barrier()
    pltpu.sync_copy(agg_s.at[pl.ds(s * OROWS, OROWS)],
                    p_hbm.at[c, pl.ds(s * OROWS, OROWS)])

    @pl.when(c == 1)
    def _():
        pltpu.sync_copy(deg_s.at[pl.ds(s * OROWS, OROWS)],
                        dg_hbm.at[pl.ds(s * OROWS, OROWS)])


_sc_agg_x = functools.partial(
    pl.kernel,
    mesh=_mesh,
    compiler_params=_sc_params,
    out_type=[jax.ShapeDtypeStruct((NC, R, D_HALF), jnp.float32),
              jax.ShapeDtypeStruct((R, D_HID), jnp.float32)],
    scratch_types=[
        pltpu.VMEM((NCH2, CH1), jnp.int32),
        pltpu.VMEM((NCH2, CH1), jnp.int32),
        pltpu.VMEM((CH1, D_HALF), jnp.float32),
        pltpu.VMEM((CH1, D_HALF), jnp.float32),
        pltpu.VMEM((CH1, D_HID), jnp.float32),
        pltpu.VMEM_SHARED((RS, D_HALF), jnp.float32),
        pltpu.VMEM_SHARED((RS, D_HID), jnp.float32),
        pltpu.SemaphoreType.DMA,
        pltpu.SemaphoreType.DMA,
        pltpu.SemaphoreType.DMA,
        pltpu.SemaphoreType.DMA,
        pltpu.SemaphoreType.DMA,
    ],
)(_sc_agg_x_body)


def _sc_agg_h_body(h_hbm, src_hbm, dst_hbm, p_hbm,
                   src_v, dst_v, rows_a, rows_b, agg_s,
                   sem_a, sem_b, sem_sa, sem_sb):
    c = lax.axis_index("c")
    s = lax.axis_index("s")
    wid = c * NS + s
    _zero_spmem(rows_a, agg_s, s, D_HID)
    pltpu.sync_copy(src_hbm.at[wid], src_v)
    pltpu.sync_copy(dst_hbm.at[wid], dst_v)
    plsc.subcore_barrier()

    def start(j, buf, sem):
        pltpu.async_copy(h_hbm.at[src_v.at[j]], buf, sem)

    def wait(j, buf, sem):
        pltpu.make_async_copy(h_hbm.at[src_v.at[j]], buf, sem).wait()

    def s_start(j, buf, sem):
        pltpu.async_copy(buf, agg_s.at[dst_v.at[j]], sem, add=True)

    def s_wait(j, buf, sem):
        pltpu.make_async_copy(buf, agg_s.at[dst_v.at[j]], sem).wait()

    start(0, rows_a, sem_a)

    def pair(i, carry):
        j0 = 2 * i
        wait(j0, rows_a, sem_a)
        start(j0 + 1, rows_b, sem_b)
        s_start(j0, rows_a, sem_sa)
        wait(j0 + 1, rows_b, sem_b)
        s_wait(j0, rows_a, sem_sa)
        start(j0 + 2, rows_a, sem_a)
        s_start(j0 + 1, rows_b, sem_sb)
        s_wait(j0 + 1, rows_b, sem_sb)
        return carry

    lax.fori_loop(0, NCH // 2 - 1, pair, 0)
    wait(NCH - 2, rows_a, sem_a)
    start(NCH - 1, rows_b, sem_b)
    s_start(NCH - 2, rows_a, sem_sa)
    wait(NCH - 1, rows_b, sem_b)
    s_wait(NCH - 2, rows_a, sem_sa)
    s_start(NCH - 1, rows_b, sem_sb)
    s_wait(NCH - 1, rows_b, sem_sb)
    plsc.subcore_barrier()
    pltpu.sync_copy(agg_s.at[pl.ds(s * OROWS, OROWS)],
                    p_hbm.at[c, pl.ds(s * OROWS, OROWS)])


_sc_agg_h = functools.partial(
    pl.kernel,
    mesh=_mesh,
    compiler_params=_sc_params,
    out_type=[jax.ShapeDtypeStruct((NC, R, D_HID), jnp.float32)],
    scratch_types=[
        pltpu.VMEM((NCH, CH2), jnp.int32),
        pltpu.VMEM((NCH, CH2), jnp.int32),
        pltpu.VMEM((CH2, D_HID), jnp.float32),
        pltpu.VMEM((CH2, D_HID), jnp.float32),
        pltpu.VMEM_SHARED((RS, D_HID), jnp.float32),
        pltpu.SemaphoreType.DMA,
        pltpu.SemaphoreType.DMA,
        pltpu.SemaphoreType.DMA,
        pltpu.SemaphoreType.DMA,
    ],
)(_sc_agg_h_body)


def _lay1_body(p_ref, d_ref, w1_ref, b1_ref, o_ref):
    agg = jnp.concatenate([p_ref[0], p_ref[1]], axis=1)
    deg = jnp.maximum(d_ref[:, 0:1], 1.0)
    o_ref[...] = jnp.maximum(
        jnp.dot(agg / deg, w1_ref[...],
                preferred_element_type=jnp.float32) + b1_ref[...], 0.0)


def _fin_body(p_ref, d_ref, w2_ref, b2_ref, batch_ref, fcw_ref, fcb_ref,
              o_ref):
    agg = p_ref[0] + p_ref[1]
    deg = jnp.maximum(d_ref[:, 0:1], 1.0)
    h2 = jnp.maximum(
        jnp.dot(agg / deg, w2_ref[...],
                preferred_element_type=jnp.float32) + b2_ref[...], 0.0)
    b = batch_ref[...]
    gid = lax.broadcasted_iota(jnp.int32, (N_GRAPHS, R), 0)
    onehot = (b == gid).astype(jnp.float32)
    sums = jnp.dot(onehot, h2, preferred_element_type=jnp.float32,
                   precision=lax.Precision.HIGHEST)
    counts = jnp.sum(onehot, axis=1, keepdims=True)
    pooled = sums / jnp.maximum(counts, 1.0)
    o_ref[...] = (jnp.dot(pooled, fcw_ref[...],
                          preferred_element_type=jnp.float32) + fcb_ref[...])


_lay1 = pl.pallas_call(
    _lay1_body, out_shape=jax.ShapeDtypeStruct((R, D_HID), jnp.float32))

_fin = pl.pallas_call(
    _fin_body, out_shape=jax.ShapeDtypeStruct((N_GRAPHS, 1), jnp.float32))


def kernel(x, edge_index, batch, W1, b1, W2, b2, fc_W, fc_b):
    src = edge_index[0].astype(jnp.int32)
    dst = edge_index[1].astype(jnp.int32)
    n_pad_e = E_PAD - N_EDGES
    src_p = jnp.concatenate([src, jnp.zeros((n_pad_e,), jnp.int32)])
    dst_p = jnp.concatenate([dst, jnp.full((n_pad_e,), TRASH, jnp.int32)])
    src2 = src_p.reshape(NS, NCH2, CH1)
    dst2 = dst_p.reshape(NS, NCH2, CH1)
    src3 = src_p.reshape(NW, NCH, CH2)
    dst3 = dst_p.reshape(NW, NCH, CH2)
    xcat = jnp.concatenate([x[:, :D_HALF], x[:, D_HALF:]], axis=0)
    batch2 = jnp.concatenate(
        [batch.astype(jnp.int32),
         jnp.full((R - N_NODES,), N_GRAPHS, jnp.int32)]).reshape(1, R)

    p1, dg = _sc_agg_x(xcat, src2, dst2)
    h1 = _lay1(p1, dg, W1, b1.reshape(1, D_HID))
    (p2,) = _sc_agg_h(h1, src3, dst3)
    out = _fin(p2, dg, W2, b2.reshape(1, D_HID), batch2, fc_W,
               fc_b.reshape(1, 1))
    return out
